# baseline (speedup 1.0000x reference)
import functools

import jax
import jax.numpy as jnp
import numpy as np
from jax import lax
from jax.experimental import pallas as pl
from jax.experimental.pallas import tpu as pltpu
import jax.experimental.pallas.tpu_sc as plsc

SIG_DIM = 32
HALF = 16
N_NODES = 100000
N_ADJ_EDGES = 1600000
N_QUERY = 65536

N_TILES = 16
LANES = 16

NODE_T = 100352
ROWS_PER_TILE = NODE_T // N_TILES
ZCHUNK = ROWS_PER_TILE // 14
DEGCH = ROWS_PER_TILE // 4

EDGE_T = 1605632
EDGES_PER_TILE = EDGE_T // N_TILES
SUPER = 512
N_SUPER = EDGES_PER_TILE // SUPER
GROUPS = SUPER // 128
REAL_GROUPS = N_ADJ_EDGES // 128
PAD_GROUPS = EDGE_T // 128 - REAL_GROUPS

Q_PER_TILE = N_QUERY // 32
Q_CHUNKS = Q_PER_TILE // 128


def _np_threefry2x32(k1, k2, x0, x1):
    def rotl(x, r):
        return ((x << np.uint32(r)) | (x >> np.uint32(32 - r))).astype(np.uint32)

    def rounds(v0, v1, rots):
        for r in rots:
            v0 = (v0 + v1).astype(np.uint32)
            v1 = v0 ^ rotl(v1, r)
        return v0, v1

    rot0 = (13, 15, 26, 6)
    rot1 = (17, 29, 16, 24)
    ks0, ks1 = np.uint32(k1), np.uint32(k2)
    ks2 = np.uint32(ks0 ^ ks1 ^ np.uint32(0x1BD11BDA))
    x0 = (x0 + ks0).astype(np.uint32)
    x1 = (x1 + ks1).astype(np.uint32)
    sched = [(ks1, ks2), (ks2, ks0), (ks0, ks1), (ks1, ks2), (ks2, ks0)]
    for i, (a, b) in enumerate(sched):
        x0, x1 = rounds(x0, x1, rot0 if i % 2 == 0 else rot1)
        x0 = (x0 + a).astype(np.uint32)
        x1 = (x1 + b + np.uint32(i + 1)).astype(np.uint32)
    return x0, x1


def _np_erfinv32(x):
    w = -np.log((np.float32(1.0) - x) * (np.float32(1.0) + x))
    lt = w < np.float32(5.0)
    wa = np.where(lt, w - np.float32(2.5),
                  np.sqrt(np.maximum(w, np.float32(5.0))) - np.float32(3.0))
    ca = [2.81022636e-08, 3.43273939e-07, -3.5233877e-06, -4.39150654e-06,
          0.00021858087, -0.00125372503, -0.00417768164, 0.246640727,
          1.50140941]
    cb = [-0.000200214257, 0.000100950558, 0.00134934322, -0.00367342844,
          0.00573950773, -0.0076224613, 0.00943887047, 1.00167406, 2.83297682]
    pa = np.full_like(wa, np.float32(ca[0]))
    pb = np.full_like(wa, np.float32(cb[0]))
    for c in ca[1:]:
        pa = np.float32(c) + pa * wa
    for c in cb[1:]:
        pb = np.float32(c) + pb * wa
    return (np.where(lt, pa, pb) * x).astype(np.float32)


def _make_sig_halves():
    size = N_NODES * SIG_DIM
    idx = np.arange(size, dtype=np.uint64)
    c1 = (idx >> np.uint64(32)).astype(np.uint32)
    c2 = (idx & np.uint64(0xFFFFFFFF)).astype(np.uint32)
    b1, b2 = _np_threefry2x32(np.uint32(0), np.uint32(42), c1, c2)
    bits = b1 ^ b2
    float_bits = (bits >> np.uint32(9)) | np.uint32(0x3F800000)
    floats = float_bits.view(np.float32) - np.float32(1.0)
    lo = np.float32(np.nextafter(np.float32(-1.0), np.float32(0.0)))
    u = np.maximum(lo, (floats * (np.float32(1.0) - lo) + lo).astype(np.float32))
    rv = (np.float32(np.sqrt(2.0)) * _np_erfinv32(u)).reshape(N_NODES, SIG_DIM)
    norm = np.maximum(np.sqrt((rv * rv).sum(1, keepdims=True, dtype=np.float32)),
                      np.float32(1e-12))
    rv = (rv / norm).astype(np.float32)
    pad = np.zeros((NODE_T - N_NODES, HALF), np.float32)
    return (np.ascontiguousarray(np.concatenate([rv[:, :HALF], pad], 0)).reshape(-1),
            np.ascontiguousarray(np.concatenate([rv[:, HALF:], pad], 0)).reshape(-1))


_SIG_LO, _SIG_HI = _make_sig_halves()


def _propagate_body(sig_lo, sig_hi, w_pad, ei, pe, z2, z1,
                    oh_lo, oh_hi, ti_lo, ti_hi, deg_hbm, x_lo, x_hi,
                    acc, deg_sh, degbuf, colv, rowv, vals,
                    ones_v, isemA, isemB, gsemA, gsemB, ssem, zsem):
    c = lax.axis_index("c")
    s = lax.axis_index("s")
    acc_base = s * ROWS_PER_TILE

    for jj in range(8):
        ones_v[pl.ds(jj * LANES, LANES)] = jnp.ones((LANES,), jnp.float32)

    def scale_sigs(sig_ref, x_ref):
        for chz in range(8):
            rb = acc_base + chz * 784
            pltpu.sync_copy(w_pad.at[pl.ds(rb, 784)], degbuf.at[pl.ds(0, 784)])
            pltpu.sync_copy(sig_ref.at[pl.ds(rb, 784)], vals.at[pl.ds(0, 784)])

            def row(r, carry):
                wv = plsc.load_gather(degbuf, [jnp.full((LANES,), r, jnp.int32)])
                vals[r] = vals[r] * wv
                return carry
            lax.fori_loop(0, 784, row, 0)
            pltpu.sync_copy(vals.at[pl.ds(0, 784)], x_ref.at[pl.ds(rb, 784)])

    def zero_acc():
        pltpu.sync_copy(z2, vals.at[pl.ds(0, ZCHUNK)])
        for i in range(14):
            pltpu.async_copy(vals.at[pl.ds(0, ZCHUNK)],
                             acc.at[pl.ds(acc_base + i * ZCHUNK, ZCHUNK)], zsem)
        for i in range(14):
            pltpu.make_async_copy(vals.at[pl.ds(0, ZCHUNK)],
                                  acc.at[pl.ds(acc_base, ZCHUNK)], zsem).wait()

    def writeback(dst_ref):
        def ld(i):
            return pltpu.async_copy(
                acc.at[pl.ds(acc_base + i * ZCHUNK, ZCHUNK)],
                vals.at[pl.ds((i % 2) * ZCHUNK, ZCHUNK)], gsemA)

        def st(i):
            return pltpu.async_copy(
                vals.at[pl.ds((i % 2) * ZCHUNK, ZCHUNK)],
                dst_ref.at[pl.ds(acc_base + i * ZCHUNK, ZCHUNK)], zsem)

        lds = [ld(0)]
        sts = []
        for i in range(14):
            lds[i].wait()
            if i >= 1:
                sts[i - 1].wait()
            if i < 13:
                lds.append(ld(i + 1))
            sts.append(st(i))
        sts[13].wait()

    def spmm(src_ref, with_deg):
        isems = (isemA, isemB)
        gsems = (gsemA, gsemB)

        def load_idx(g, slot):
            gbase = s * (EDGES_PER_TILE // 128) + g * GROUPS
            sl = pl.ds(slot * GROUPS, GROUPS)

            @pl.when(gbase < REAL_GROUPS)
            def _():
                pltpu.async_copy(ei.at[1, pl.ds(gbase, GROUPS)], colv.at[sl],
                                 isems[slot])
                pltpu.async_copy(ei.at[0, pl.ds(gbase, GROUPS)], rowv.at[sl],
                                 isems[slot])

            @pl.when(gbase >= REAL_GROUPS)
            def _():
                pb = gbase - REAL_GROUPS
                pltpu.async_copy(pe.at[1, pl.ds(pb, GROUPS)], colv.at[sl],
                                 isems[slot])
                pltpu.async_copy(pe.at[0, pl.ds(pb, GROUPS)], rowv.at[sl],
                                 isems[slot])

        def drain_idx(slot):
            sl = pl.ds(slot * GROUPS, GROUPS)
            pltpu.make_async_copy(ei.at[1, pl.ds(0, GROUPS)], colv.at[sl],
                                  isems[slot]).wait()
            pltpu.make_async_copy(ei.at[0, pl.ds(0, GROUPS)], rowv.at[sl],
                                  isems[slot]).wait()

        def fire_gathers(slot):
            for j in range(GROUPS):
                r = slot * GROUPS + j
                pltpu.async_copy(src_ref.at[colv.at[r]],
                                 vals.at[pl.ds(r * 128, 128)], gsems[slot])

        def drain_gathers(slot):
            for j in range(GROUPS):
                r = slot * GROUPS + j
                pltpu.make_async_copy(src_ref.at[pl.ds(0, 128)],
                                      vals.at[pl.ds(r * 128, 128)],
                                      gsems[slot]).wait()

        def fire_scatters(slot):
            for j in range(GROUPS):
                r = slot * GROUPS + j
                pltpu.async_copy(vals.at[pl.ds(r * 128, 128)],
                                 acc.at[rowv.at[r]], ssem, add=True)
                if with_deg:
                    pltpu.async_copy(ones_v, deg_sh.at[rowv.at[r]], ssem,
                                     add=True)

        def drain_scatters(slot):
            for j in range(GROUPS):
                r = slot * GROUPS + j
                pltpu.make_async_copy(src_ref.at[pl.ds(0, 128)],
                                      vals.at[pl.ds(r * 128, 128)],
                                      ssem).wait()
                if with_deg:
                    pltpu.make_async_copy(deg_hbm.at[pl.ds(0, 128)],
                                          degbuf.at[pl.ds(0, 128)],
                                          ssem).wait()

        load_idx(0, 0)
        drain_idx(0)
        fire_gathers(0)
        load_idx(1, 1)

        def body(b, carry):
            drain_gathers(0)
            fire_scatters(0)
            drain_idx(1)
            fire_gathers(1)
            drain_scatters(0)

            @pl.when(b < N_SUPER // 2 - 1)
            def _():
                load_idx(2 * b + 2, 0)
            drain_gathers(1)
            fire_scatters(1)

            @pl.when(b < N_SUPER // 2 - 1)
            def _():
                drain_idx(0)
                fire_gathers(0)
            drain_scatters(1)

            @pl.when(b < N_SUPER // 2 - 1)
            def _():
                load_idx(2 * b + 3, 1)
            return carry
        lax.fori_loop(0, N_SUPER // 2, body, 0)

    def half(sig_ref, x_ref, oh_ref, ti_ref, do_deg):
        with jax.named_scope("p_scale"):
            scale_sigs(sig_ref, x_ref)
            if do_deg:
                pltpu.sync_copy(z1, degbuf)
                for i in range(4):
                    pltpu.sync_copy(
                        degbuf, deg_sh.at[pl.ds(acc_base + i * DEGCH, DEGCH)])
            zero_acc()
            plsc.subcore_barrier()
        with jax.named_scope("p_spmm1"):
            spmm(x_ref, with_deg=do_deg)
            plsc.subcore_barrier()
        with jax.named_scope("p_wb1"):
            writeback(oh_ref)
            zero_acc()
            plsc.subcore_barrier()
        with jax.named_scope("p_spmm2"):
            spmm(oh_ref, with_deg=False)
            plsc.subcore_barrier()
        with jax.named_scope("p_wb2"):
            writeback(ti_ref)
            if do_deg:
                for i in range(4):
                    sl = pl.ds(acc_base + i * DEGCH, DEGCH)
                    pltpu.sync_copy(deg_sh.at[sl], degbuf)
                    pltpu.sync_copy(degbuf, deg_hbm.at[sl])

    @pl.when(c == 0)
    def _():
        half(sig_lo, x_lo, oh_lo, ti_lo, True)

    @pl.when(c == 1)
    def _():
        half(sig_hi, x_hi, oh_hi, ti_hi, False)


def _feature_body(oh_lo, oh_hi, ti_lo, ti_hi, x_lo, x_hi, deg_hbm,
                  qu_g, qv_g, feat_t,
                  qbuf, bufs, dgu, dgv, fbuf, gsem):
    c = lax.axis_index("c")
    s = lax.axis_index("s")
    wid = s * 2 + c

    def chunk(ch, carry):
        qrow = wid * Q_CHUNKS + ch
        pltpu.sync_copy(qu_g.at[pl.ds(qrow, 1)], qbuf.at[pl.ds(0, 1)])
        pltpu.sync_copy(qv_g.at[pl.ds(qrow, 1)], qbuf.at[pl.ds(1, 1)])
        ui = qbuf.at[0]
        vi = qbuf.at[1]
        cps = []
        for k, (tab, idx) in enumerate((
                (oh_lo, ui), (oh_hi, ui), (ti_lo, ui),
                (x_lo, ui), (x_hi, ui),
                (oh_lo, vi), (oh_hi, vi), (ti_lo, vi),
                (x_lo, vi), (x_hi, vi), (ti_hi, ui), (ti_hi, vi))):
            cps.append(pltpu.async_copy(tab.at[idx], bufs.at[k], gsem))
        cps.append(pltpu.async_copy(deg_hbm.at[ui], dgu, gsem))
        cps.append(pltpu.async_copy(deg_hbm.at[vi], dgv, gsem))
        for cp in cps:
            cp.wait()

        def group(g16, carry2):
            e0 = g16 * LANES
            ev = e0 + lax.iota(jnp.int32, LANES)
            du = dgu[pl.ds(e0, LANES)]
            dv = dgv[pl.ds(e0, LANES)]

            def gat(k, dvec):
                return plsc.load_gather(bufs.at[k], [ev, dvec])

            z = jnp.zeros((LANES,), jnp.float32)
            s11 = s12a = s12b = s22 = z
            n1u = n1v = n2u = n2v = z
            for d in range(SIG_DIM):
                dvec = jnp.full((LANES,), d % HALF, jnp.int32)
                if d < HALF:
                    k1, kt_u, kx_u, kv1, kt_v, kx_v = 0, 2, 3, 5, 7, 8
                else:
                    k1, kt_u, kx_u, kv1, kt_v, kx_v = 1, 10, 4, 6, 11, 9
                a1 = gat(k1, dvec)
                b1 = gat(kv1, dvec)
                a2 = gat(kt_u, dvec) - du * gat(kx_u, dvec)
                b2 = gat(kt_v, dvec) - dv * gat(kx_v, dvec)
                s11 = s11 + a1 * b1
                s12a = s12a + a1 * b2
                s12b = s12b + a2 * b1
                s22 = s22 + a2 * b2
                n1u = n1u + a1 * a1
                n1v = n1v + b1 * b1
                n2u = n2u + a2 * a2
                n2v = n2v + b2 * b2
            l12 = s12a + s12b
            sl = pl.ds(e0, LANES)
            fbuf[0, sl] = s11
            fbuf[1, sl] = l12
            fbuf[2, sl] = s22
            fbuf[3, sl] = n1u + n1v - 2.0 * s11 - l12
            fbuf[4, sl] = n2u + n2v - 2.0 * s22 - l12
            return carry2
        lax.fori_loop(0, 128 // LANES, group, 0)
        pltpu.sync_copy(
            fbuf, feat_t.at[:, pl.ds(wid * Q_PER_TILE + ch * 128, 128)])
        return carry
    lax.fori_loop(0, Q_CHUNKS, chunk, 0)


def _build_kernels(interpret=False):
    mesh = plsc.VectorSubcoreMesh(core_axis_name="c", subcore_axis_name="s",
                                  num_cores=2, num_subcores=N_TILES)
    f32 = jnp.float32
    cparams = pltpu.CompilerParams(use_tc_tiling_on_sc=False,
                                   needs_layout_passes=False)
    k1 = pl.kernel(
        _propagate_body,
        out_type=(
            jax.ShapeDtypeStruct((NODE_T, HALF), f32),
            jax.ShapeDtypeStruct((NODE_T, HALF), f32),
            jax.ShapeDtypeStruct((NODE_T, HALF), f32),
            jax.ShapeDtypeStruct((NODE_T, HALF), f32),
            jax.ShapeDtypeStruct((NODE_T,), f32),
            jax.ShapeDtypeStruct((NODE_T, HALF), f32),
            jax.ShapeDtypeStruct((NODE_T, HALF), f32),
        ),
        mesh=mesh,
        scratch_types=[
            pltpu.VMEM_SHARED((NODE_T, HALF), f32),
            pltpu.VMEM_SHARED((NODE_T,), f32),
            pltpu.VMEM((DEGCH,), f32),
            pltpu.VMEM((2 * GROUPS, 128), jnp.int32),
            pltpu.VMEM((2 * GROUPS, 128), jnp.int32),
            pltpu.VMEM((2 * SUPER, HALF), f32),
            pltpu.VMEM((128,), f32),
            pltpu.SemaphoreType.DMA,
            pltpu.SemaphoreType.DMA,
            pltpu.SemaphoreType.DMA,
            pltpu.SemaphoreType.DMA,
            pltpu.SemaphoreType.DMA,
            pltpu.SemaphoreType.DMA,
        ],
        compiler_params=cparams,
        interpret=interpret,
    )
    k2 = pl.kernel(
        _feature_body,
        out_type=jax.ShapeDtypeStruct((5, N_QUERY), f32),
        mesh=mesh,
        scratch_types=[
            pltpu.VMEM((2, 128), jnp.int32),
            pltpu.VMEM((12, 128, HALF), f32),
            pltpu.VMEM((128,), f32),
            pltpu.VMEM((128,), f32),
            pltpu.VMEM((5, 128), f32),
            pltpu.SemaphoreType.DMA,
        ],
        compiler_params=cparams,
        interpret=interpret,
    )
    return k1, k2


_K1, _K2 = _build_kernels()


@jax.jit
def kernel(edges, edge_index, node_weight):
    wp = jnp.pad(node_weight, (0, NODE_T - N_NODES))
    ei = edge_index.reshape(2, REAL_GROUPS, 128)
    pe = jnp.stack([jnp.full((PAD_GROUPS, 128), N_NODES, jnp.int32),
                    jnp.zeros((PAD_GROUPS, 128), jnp.int32)])
    z2 = jnp.zeros((ZCHUNK, HALF), jnp.float32)
    z1 = jnp.zeros((DEGCH,), jnp.float32)
    oh_lo, oh_hi, ti_lo, ti_hi, deg, x_lo, x_hi = _K1(
        jnp.asarray(_SIG_LO).reshape(NODE_T, HALF),
        jnp.asarray(_SIG_HI).reshape(NODE_T, HALF), wp, ei, pe, z2, z1)
    qu_g = edges[0].reshape(-1, 128)
    qv_g = edges[1].reshape(-1, 128)
    feat_t = _K2(oh_lo, oh_hi, ti_lo, ti_hi, x_lo, x_hi, deg, qu_g, qv_g)
    return feat_t.T


# speedup vs baseline: 1.0432x; 1.0432x over previous
import functools

import jax
import jax.numpy as jnp
import numpy as np
from jax import lax
from jax.experimental import pallas as pl
from jax.experimental.pallas import tpu as pltpu
import jax.experimental.pallas.tpu_sc as plsc

SIG_DIM = 32
HALF = 16
N_NODES = 100000
N_ADJ_EDGES = 1600000
N_QUERY = 65536

N_TILES = 16
LANES = 16

NODE_T = 100352
ROWS_PER_TILE = NODE_T // N_TILES
ZCHUNK = ROWS_PER_TILE // 14
DEGCH = ROWS_PER_TILE // 4

EDGE_T = 1605632
EDGES_PER_TILE = EDGE_T // N_TILES
SUPER = 512
N_SUPER = EDGES_PER_TILE // SUPER
GROUPS = SUPER // 128
REAL_GROUPS = N_ADJ_EDGES // 128
PAD_GROUPS = EDGE_T // 128 - REAL_GROUPS

Q_PER_TILE = N_QUERY // 32
Q_CHUNKS = Q_PER_TILE // 128


def _np_threefry2x32(k1, k2, x0, x1):
    def rotl(x, r):
        return ((x << np.uint32(r)) | (x >> np.uint32(32 - r))).astype(np.uint32)

    def rounds(v0, v1, rots):
        for r in rots:
            v0 = (v0 + v1).astype(np.uint32)
            v1 = v0 ^ rotl(v1, r)
        return v0, v1

    rot0 = (13, 15, 26, 6)
    rot1 = (17, 29, 16, 24)
    ks0, ks1 = np.uint32(k1), np.uint32(k2)
    ks2 = np.uint32(ks0 ^ ks1 ^ np.uint32(0x1BD11BDA))
    x0 = (x0 + ks0).astype(np.uint32)
    x1 = (x1 + ks1).astype(np.uint32)
    sched = [(ks1, ks2), (ks2, ks0), (ks0, ks1), (ks1, ks2), (ks2, ks0)]
    for i, (a, b) in enumerate(sched):
        x0, x1 = rounds(x0, x1, rot0 if i % 2 == 0 else rot1)
        x0 = (x0 + a).astype(np.uint32)
        x1 = (x1 + b + np.uint32(i + 1)).astype(np.uint32)
    return x0, x1


def _np_erfinv32(x):
    w = -np.log((np.float32(1.0) - x) * (np.float32(1.0) + x))
    lt = w < np.float32(5.0)
    wa = np.where(lt, w - np.float32(2.5),
                  np.sqrt(np.maximum(w, np.float32(5.0))) - np.float32(3.0))
    ca = [2.81022636e-08, 3.43273939e-07, -3.5233877e-06, -4.39150654e-06,
          0.00021858087, -0.00125372503, -0.00417768164, 0.246640727,
          1.50140941]
    cb = [-0.000200214257, 0.000100950558, 0.00134934322, -0.00367342844,
          0.00573950773, -0.0076224613, 0.00943887047, 1.00167406, 2.83297682]
    pa = np.full_like(wa, np.float32(ca[0]))
    pb = np.full_like(wa, np.float32(cb[0]))
    for c in ca[1:]:
        pa = np.float32(c) + pa * wa
    for c in cb[1:]:
        pb = np.float32(c) + pb * wa
    return (np.where(lt, pa, pb) * x).astype(np.float32)


def _make_sig_halves():
    size = N_NODES * SIG_DIM
    idx = np.arange(size, dtype=np.uint64)
    c1 = (idx >> np.uint64(32)).astype(np.uint32)
    c2 = (idx & np.uint64(0xFFFFFFFF)).astype(np.uint32)
    b1, b2 = _np_threefry2x32(np.uint32(0), np.uint32(42), c1, c2)
    bits = b1 ^ b2
    float_bits = (bits >> np.uint32(9)) | np.uint32(0x3F800000)
    floats = float_bits.view(np.float32) - np.float32(1.0)
    lo = np.float32(np.nextafter(np.float32(-1.0), np.float32(0.0)))
    u = np.maximum(lo, (floats * (np.float32(1.0) - lo) + lo).astype(np.float32))
    rv = (np.float32(np.sqrt(2.0)) * _np_erfinv32(u)).reshape(N_NODES, SIG_DIM)
    norm = np.maximum(np.sqrt((rv * rv).sum(1, keepdims=True, dtype=np.float32)),
                      np.float32(1e-12))
    rv = (rv / norm).astype(np.float32)
    pad = np.zeros((NODE_T - N_NODES, HALF), np.float32)
    return (np.ascontiguousarray(np.concatenate([rv[:, :HALF], pad], 0)).reshape(-1),
            np.ascontiguousarray(np.concatenate([rv[:, HALF:], pad], 0)).reshape(-1))


_SIG_LO, _SIG_HI = _make_sig_halves()


def _propagate_body(x_lo, x_hi, ei, pe, z2, z1,
                    oh_lo, oh_hi, ti_lo, ti_hi, deg_hbm,
                    acc, deg_sh, degbuf, colv, rowv, vals,
                    ones_v, isemA, isemB, gsemA, gsemB, ssem, zsem):
    c = lax.axis_index("c")
    s = lax.axis_index("s")
    acc_base = s * ROWS_PER_TILE

    for jj in range(8):
        ones_v[pl.ds(jj * LANES, LANES)] = jnp.ones((LANES,), jnp.float32)

    def zero_acc():
        pltpu.sync_copy(z2, vals.at[pl.ds(0, ZCHUNK)])
        for i in range(14):
            pltpu.async_copy(vals.at[pl.ds(0, ZCHUNK)],
                             acc.at[pl.ds(acc_base + i * ZCHUNK, ZCHUNK)], zsem)
        for i in range(14):
            pltpu.make_async_copy(vals.at[pl.ds(0, ZCHUNK)],
                                  acc.at[pl.ds(acc_base, ZCHUNK)], zsem).wait()

    def writeback(dst_ref):
        def ld(i):
            return pltpu.async_copy(
                acc.at[pl.ds(acc_base + i * ZCHUNK, ZCHUNK)],
                vals.at[pl.ds((i % 2) * ZCHUNK, ZCHUNK)], gsemA)

        def st(i):
            return pltpu.async_copy(
                vals.at[pl.ds((i % 2) * ZCHUNK, ZCHUNK)],
                dst_ref.at[pl.ds(acc_base + i * ZCHUNK, ZCHUNK)], zsem)

        lds = [ld(0)]
        sts = []
        for i in range(14):
            lds[i].wait()
            if i >= 1:
                sts[i - 1].wait()
            if i < 13:
                lds.append(ld(i + 1))
            sts.append(st(i))
        sts[13].wait()

    def spmm(src_ref, with_deg):
        isems = (isemA, isemB)
        gsems = (gsemA, gsemB)

        def load_idx(g, slot):
            gbase = s * (EDGES_PER_TILE // 128) + g * GROUPS
            sl = pl.ds(slot * GROUPS, GROUPS)

            @pl.when(gbase < REAL_GROUPS)
            def _():
                pltpu.async_copy(ei.at[1, pl.ds(gbase, GROUPS)], colv.at[sl],
                                 isems[slot])
                pltpu.async_copy(ei.at[0, pl.ds(gbase, GROUPS)], rowv.at[sl],
                                 isems[slot])

            @pl.when(gbase >= REAL_GROUPS)
            def _():
                pb = gbase - REAL_GROUPS
                pltpu.async_copy(pe.at[1, pl.ds(pb, GROUPS)], colv.at[sl],
                                 isems[slot])
                pltpu.async_copy(pe.at[0, pl.ds(pb, GROUPS)], rowv.at[sl],
                                 isems[slot])

        def drain_idx(slot):
            sl = pl.ds(slot * GROUPS, GROUPS)
            pltpu.make_async_copy(ei.at[1, pl.ds(0, GROUPS)], colv.at[sl],
                                  isems[slot]).wait()
            pltpu.make_async_copy(ei.at[0, pl.ds(0, GROUPS)], rowv.at[sl],
                                  isems[slot]).wait()

        def fire_gathers(slot):
            for j in range(GROUPS):
                r = slot * GROUPS + j
                pltpu.async_copy(src_ref.at[colv.at[r]],
                                 vals.at[pl.ds(r * 128, 128)], gsems[slot])

        def drain_gathers(slot):
            for j in range(GROUPS):
                r = slot * GROUPS + j
                pltpu.make_async_copy(src_ref.at[pl.ds(0, 128)],
                                      vals.at[pl.ds(r * 128, 128)],
                                      gsems[slot]).wait()

        def fire_scatters(slot):
            for j in range(GROUPS):
                r = slot * GROUPS + j
                pltpu.async_copy(vals.at[pl.ds(r * 128, 128)],
                                 acc.at[rowv.at[r]], ssem, add=True)
                if with_deg:
                    pltpu.async_copy(ones_v, deg_sh.at[rowv.at[r]], ssem,
                                     add=True)

        def drain_scatters(slot):
            for j in range(GROUPS):
                r = slot * GROUPS + j
                pltpu.make_async_copy(src_ref.at[pl.ds(0, 128)],
                                      vals.at[pl.ds(r * 128, 128)],
                                      ssem).wait()
                if with_deg:
                    pltpu.make_async_copy(deg_hbm.at[pl.ds(0, 128)],
                                          degbuf.at[pl.ds(0, 128)],
                                          ssem).wait()

        load_idx(0, 0)
        drain_idx(0)
        fire_gathers(0)
        load_idx(1, 1)

        def body(b, carry):
            drain_gathers(0)
            fire_scatters(0)
            drain_idx(1)
            fire_gathers(1)
            drain_scatters(0)

            @pl.when(b < N_SUPER // 2 - 1)
            def _():
                load_idx(2 * b + 2, 0)
            drain_gathers(1)
            fire_scatters(1)

            @pl.when(b < N_SUPER // 2 - 1)
            def _():
                drain_idx(0)
                fire_gathers(0)
            drain_scatters(1)

            @pl.when(b < N_SUPER // 2 - 1)
            def _():
                load_idx(2 * b + 3, 1)
            return carry
        lax.fori_loop(0, N_SUPER // 2, body, 0)

    def half(x_ref, oh_ref, ti_ref, do_deg):
        with jax.named_scope("p_scale"):
            if do_deg:
                pltpu.sync_copy(z1, degbuf)
                for i in range(4):
                    pltpu.sync_copy(
                        degbuf, deg_sh.at[pl.ds(acc_base + i * DEGCH, DEGCH)])
            zero_acc()
            plsc.subcore_barrier()
        with jax.named_scope("p_spmm1"):
            spmm(x_ref, with_deg=do_deg)
            plsc.subcore_barrier()
        with jax.named_scope("p_wb1"):
            writeback(oh_ref)
            zero_acc()
            plsc.subcore_barrier()
        with jax.named_scope("p_spmm2"):
            spmm(oh_ref, with_deg=False)
            plsc.subcore_barrier()
        with jax.named_scope("p_wb2"):
            writeback(ti_ref)
            if do_deg:
                for i in range(4):
                    sl = pl.ds(acc_base + i * DEGCH, DEGCH)
                    pltpu.sync_copy(deg_sh.at[sl], degbuf)
                    pltpu.sync_copy(degbuf, deg_hbm.at[sl])

    @pl.when(c == 0)
    def _():
        half(x_lo, oh_lo, ti_lo, True)

    @pl.when(c == 1)
    def _():
        half(x_hi, oh_hi, ti_hi, False)


def _feature_body(oh_lo, oh_hi, ti_lo, ti_hi, x_lo, x_hi, deg_hbm,
                  qu_g, qv_g, feat_t,
                  qallu, qallv, bufs, dgu, dgv, fbuf,
                  gsemA, gsemB, fsem):
    c = lax.axis_index("c")
    s = lax.axis_index("s")
    wid = s * 2 + c
    gsems = (gsemA, gsemB)

    pltpu.sync_copy(qu_g.at[pl.ds(wid * Q_CHUNKS, Q_CHUNKS)], qallu)
    pltpu.sync_copy(qv_g.at[pl.ds(wid * Q_CHUNKS, Q_CHUNKS)], qallv)

    def fire(ch, slot):
        ui = qallu.at[ch]
        vi = qallv.at[ch]
        for k, (tab, idx) in enumerate((
                (oh_lo, ui), (oh_hi, ui), (ti_lo, ui),
                (x_lo, ui), (x_hi, ui),
                (oh_lo, vi), (oh_hi, vi), (ti_lo, vi),
                (x_lo, vi), (x_hi, vi), (ti_hi, ui), (ti_hi, vi))):
            pltpu.async_copy(tab.at[idx], bufs.at[slot * 12 + k], gsems[slot])
        pltpu.async_copy(deg_hbm.at[ui], dgu.at[slot], gsems[slot])
        pltpu.async_copy(deg_hbm.at[vi], dgv.at[slot], gsems[slot])

    def drain(slot):
        for k in range(12):
            pltpu.make_async_copy(oh_lo.at[pl.ds(0, 128)],
                                  bufs.at[slot * 12 + k], gsems[slot]).wait()
        pltpu.make_async_copy(deg_hbm.at[pl.ds(0, 128)], dgu.at[slot],
                              gsems[slot]).wait()
        pltpu.make_async_copy(deg_hbm.at[pl.ds(0, 128)], dgv.at[slot],
                              gsems[slot]).wait()

    def compute(ch, slot):
        def group(g16, carry2):
            e0 = g16 * LANES
            ev = e0 + lax.iota(jnp.int32, LANES)
            du = dgu[slot, pl.ds(e0, LANES)]
            dv = dgv[slot, pl.ds(e0, LANES)]

            def gat(k, dvec):
                return plsc.load_gather(bufs.at[slot * 12 + k], [ev, dvec])

            z = jnp.zeros((LANES,), jnp.float32)
            s11 = s12a = s12b = s22 = z
            n1u = n1v = n2u = n2v = z
            for d in range(SIG_DIM):
                dvec = jnp.full((LANES,), d % HALF, jnp.int32)
                if d < HALF:
                    k1, kt_u, kx_u, kv1, kt_v, kx_v = 0, 2, 3, 5, 7, 8
                else:
                    k1, kt_u, kx_u, kv1, kt_v, kx_v = 1, 10, 4, 6, 11, 9
                a1 = gat(k1, dvec)
                b1 = gat(kv1, dvec)
                a2 = gat(kt_u, dvec) - du * gat(kx_u, dvec)
                b2 = gat(kt_v, dvec) - dv * gat(kx_v, dvec)
                s11 = s11 + a1 * b1
                s12a = s12a + a1 * b2
                s12b = s12b + a2 * b1
                s22 = s22 + a2 * b2
                n1u = n1u + a1 * a1
                n1v = n1v + b1 * b1
                n2u = n2u + a2 * a2
                n2v = n2v + b2 * b2
            l12 = s12a + s12b
            sl = pl.ds(e0, LANES)
            fbuf[slot, 0, sl] = s11
            fbuf[slot, 1, sl] = l12
            fbuf[slot, 2, sl] = s22
            fbuf[slot, 3, sl] = n1u + n1v - 2.0 * s11 - l12
            fbuf[slot, 4, sl] = n2u + n2v - 2.0 * s22 - l12
            return carry2
        lax.fori_loop(0, 128 // LANES, group, 0)
        return pltpu.async_copy(
            fbuf.at[slot],
            feat_t.at[:, pl.ds(wid * Q_PER_TILE + ch * 128, 128)], fsem)

    fire(0, 0)

    def body(cb, carry):
        drain(0)
        fire(2 * cb + 1, 1)
        st0 = compute(2 * cb, 0)
        drain(1)

        @pl.when(cb < Q_CHUNKS // 2 - 1)
        def _():
            fire(2 * cb + 2, 0)
        st1 = compute(2 * cb + 1, 1)
        st0.wait()
        st1.wait()
        return carry
    lax.fori_loop(0, Q_CHUNKS // 2, body, 0)


def _build_kernels(interpret=False):
    mesh = plsc.VectorSubcoreMesh(core_axis_name="c", subcore_axis_name="s",
                                  num_cores=2, num_subcores=N_TILES)
    f32 = jnp.float32
    cparams = pltpu.CompilerParams(use_tc_tiling_on_sc=False,
                                   needs_layout_passes=False)
    k1 = pl.kernel(
        _propagate_body,
        out_type=(
            jax.ShapeDtypeStruct((NODE_T, HALF), f32),
            jax.ShapeDtypeStruct((NODE_T, HALF), f32),
            jax.ShapeDtypeStruct((NODE_T, HALF), f32),
            jax.ShapeDtypeStruct((NODE_T, HALF), f32),
            jax.ShapeDtypeStruct((NODE_T,), f32),
        ),
        mesh=mesh,
        scratch_types=[
            pltpu.VMEM_SHARED((NODE_T, HALF), f32),
            pltpu.VMEM_SHARED((NODE_T,), f32),
            pltpu.VMEM((DEGCH,), f32),
            pltpu.VMEM((2 * GROUPS, 128), jnp.int32),
            pltpu.VMEM((2 * GROUPS, 128), jnp.int32),
            pltpu.VMEM((2 * SUPER, HALF), f32),
            pltpu.VMEM((128,), f32),
            pltpu.SemaphoreType.DMA,
            pltpu.SemaphoreType.DMA,
            pltpu.SemaphoreType.DMA,
            pltpu.SemaphoreType.DMA,
            pltpu.SemaphoreType.DMA,
            pltpu.SemaphoreType.DMA,
        ],
        compiler_params=cparams,
        interpret=interpret,
    )
    k2 = pl.kernel(
        _feature_body,
        out_type=jax.ShapeDtypeStruct((5, N_QUERY), f32),
        mesh=mesh,
        scratch_types=[
            pltpu.VMEM((Q_CHUNKS, 128), jnp.int32),
            pltpu.VMEM((Q_CHUNKS, 128), jnp.int32),
            pltpu.VMEM((24, 128, HALF), f32),
            pltpu.VMEM((2, 128), f32),
            pltpu.VMEM((2, 128), f32),
            pltpu.VMEM((2, 5, 128), f32),
            pltpu.SemaphoreType.DMA,
            pltpu.SemaphoreType.DMA,
            pltpu.SemaphoreType.DMA,
        ],
        compiler_params=cparams,
        interpret=interpret,
    )
    return k1, k2


_K1, _K2 = _build_kernels()


@jax.jit
def kernel(edges, edge_index, node_weight):
    wp = jnp.pad(node_weight, (0, NODE_T - N_NODES))
    wrep = jnp.broadcast_to(wp[:, None], (NODE_T, HALF)).reshape(-1)
    x_lo = (jnp.asarray(_SIG_LO) * wrep).reshape(NODE_T, HALF)
    x_hi = (jnp.asarray(_SIG_HI) * wrep).reshape(NODE_T, HALF)
    ei = edge_index.reshape(2, REAL_GROUPS, 128)
    pe = jnp.stack([jnp.full((PAD_GROUPS, 128), N_NODES, jnp.int32),
                    jnp.zeros((PAD_GROUPS, 128), jnp.int32)])
    z2 = jnp.zeros((ZCHUNK, HALF), jnp.float32)
    z1 = jnp.zeros((DEGCH,), jnp.float32)
    oh_lo, oh_hi, ti_lo, ti_hi, deg = _K1(x_lo, x_hi, ei, pe, z2, z1)
    qu_g = edges[0].reshape(-1, 128)
    qv_g = edges[1].reshape(-1, 128)
    feat_t = _K2(oh_lo, oh_hi, ti_lo, ti_hi, x_lo, x_hi, deg, qu_g, qv_g)
    return feat_t.T
